# baseline (speedup 1.0000x reference)
import jax
import jax.numpy as jnp
from jax.experimental import pallas as pl
from jax.experimental.pallas import tpu as pltpu

_G = 8


def _make_body(bs):
    def body(emb_ref, out_ref, rep_ref, fill_sem, sem):
        pltpu.make_async_copy(emb_ref, rep_ref.at[0], fill_sem).start()
        pltpu.make_async_copy(emb_ref, rep_ref.at[0], fill_sem).wait()
        have = 1
        while have < _G:
            n = min(have, _G - have)
            c = pltpu.make_async_copy(
                rep_ref.at[pl.ds(0, n)], rep_ref.at[pl.ds(have, n)], fill_sem
            )
            c.start()
            c.wait()
            have += n

        copies = [
            pltpu.make_async_copy(rep_ref.at[b % _G], out_ref.at[b], sem)
            for b in range(bs)
        ]
        for c in copies:
            c.start()
        for c in copies:
            c.wait()

    return body


def kernel(mask, embed_weight):
    bs, t = mask.shape
    n_embed, d = embed_weight.shape

    out = pl.pallas_call(
        _make_body(bs),
        in_specs=[pl.BlockSpec(memory_space=pltpu.MemorySpace.VMEM)],
        out_specs=pl.BlockSpec(memory_space=pl.ANY),
        out_shape=jax.ShapeDtypeStruct((bs, t, d), embed_weight.dtype),
        scratch_shapes=[
            pltpu.VMEM((_G, t, d), embed_weight.dtype),
            pltpu.SemaphoreType.DMA,
            pltpu.SemaphoreType.DMA,
        ],
    )(embed_weight[:t])
    return out


# speedup vs baseline: 1.0504x; 1.0504x over previous
import jax
import jax.numpy as jnp
from jax.experimental import pallas as pl
from jax.experimental.pallas import tpu as pltpu

_G = 4


def _make_body(bs):
    def body(emb_ref, out_ref, rep_ref, fill_sem, sem):
        pltpu.make_async_copy(emb_ref, rep_ref.at[0], fill_sem).start()
        pltpu.make_async_copy(emb_ref, rep_ref.at[0], fill_sem).wait()
        have = 1
        while have < _G:
            n = min(have, _G - have)
            c = pltpu.make_async_copy(
                rep_ref.at[pl.ds(0, n)], rep_ref.at[pl.ds(have, n)], fill_sem
            )
            c.start()
            c.wait()
            have += n

        copies = [
            pltpu.make_async_copy(rep_ref, out_ref.at[pl.ds(i * _G, _G)], sem)
            for i in range(bs // _G)
        ]
        for c in copies:
            c.start()
        for c in copies:
            c.wait()

    return body


def kernel(mask, embed_weight):
    bs, t = mask.shape
    n_embed, d = embed_weight.shape

    out = pl.pallas_call(
        _make_body(bs),
        in_specs=[pl.BlockSpec(memory_space=pltpu.MemorySpace.VMEM)],
        out_specs=pl.BlockSpec(memory_space=pl.ANY),
        out_shape=jax.ShapeDtypeStruct((bs, t, d), embed_weight.dtype),
        scratch_shapes=[
            pltpu.VMEM((_G, t, d), embed_weight.dtype),
            pltpu.SemaphoreType.DMA,
            pltpu.SemaphoreType.DMA,
        ],
    )(embed_weight[:t])
    return out
